# Initial kernel scaffold; baseline (speedup 1.0000x reference)
import functools

import jax
import jax.numpy as jnp
import numpy as np
from jax import lax
from jax.experimental import pallas as pl
from jax.experimental.pallas import tpu as pltpu
from jax.experimental.pallas import tpu_sc as plsc

B = 64
MAXLEN = 2048
D_MODEL = 768
VOCAB = 5

NC = 2
NS = 16
LANES = 16
NW = NC * NS

TOKENS = B * MAXLEN
T_PER_W = TOKENS // NW
CHUNK = 128
N_CHUNKS = T_PER_W // CHUNK


def _pe_scaled() -> np.ndarray:
    pos = np.arange(MAXLEN)[:, np.newaxis]
    i = np.arange(D_MODEL)[np.newaxis, :]
    angle_rates = 1.0 / np.power(10000, 2 * (i // 2) / np.float32(D_MODEL))
    angle_rads = pos * angle_rates
    angle_rads[:, 0::2] = np.sin(angle_rads[:, 0::2])
    angle_rads[:, 1::2] = np.cos(angle_rads[:, 1::2])
    return (angle_rads / np.sqrt(np.float32(D_MODEL))).astype(np.float32)


_PE_SCALED = _pe_scaled()


def _fuse_body(table_ref, pe_ref, out_ref):
    out_ref[...] = table_ref[...][:, None, :] + pe_ref[...][None, :, :]


_L_BLK = 256


def _build_fused(table):
    pe = jnp.asarray(_PE_SCALED)
    return pl.pallas_call(
        _fuse_body,
        grid=(MAXLEN // _L_BLK,),
        in_specs=[
            pl.BlockSpec((VOCAB, D_MODEL), lambda i: (0, 0)),
            pl.BlockSpec((_L_BLK, D_MODEL), lambda i: (i, 0)),
        ],
        out_specs=pl.BlockSpec((VOCAB, _L_BLK, D_MODEL), lambda i: (0, i, 0)),
        out_shape=jax.ShapeDtypeStruct((VOCAB, MAXLEN, D_MODEL), jnp.float32),
    )(table, pe)


@functools.partial(
    pl.kernel,
    out_type=jax.ShapeDtypeStruct((TOKENS, D_MODEL), jnp.float32),
    mesh=plsc.VectorSubcoreMesh(core_axis_name="c", subcore_axis_name="s"),
    scratch_types=[
        pltpu.VMEM((CHUNK,), jnp.int32),
        pltpu.VMEM((CHUNK, D_MODEL), jnp.float32),
        pltpu.SemaphoreType.DMA,
    ],
)
def _sc_gather(tok_hbm, fused_hbm, out_hbm, idx_v, rows_v, sem):
    wid = lax.axis_index("s") * NC + lax.axis_index("c")
    base = wid * T_PER_W

    def chunk(i, carry):
        t0 = base + i * CHUNK
        pltpu.sync_copy(tok_hbm.at[pl.ds(t0, CHUNK)], idx_v)
        for r in range(CHUNK // LANES):
            sl = pl.ds(r * LANES, LANES)
            tok16 = idx_v[sl]
            t16 = (t0 + r * LANES) + lax.iota(jnp.int32, (LANES,))
            idx_v[sl] = (tok16 << 11) | (t16 & (MAXLEN - 1))
        pltpu.async_copy(fused_hbm.at[idx_v], rows_v, sem).wait()
        pltpu.sync_copy(rows_v, out_hbm.at[pl.ds(t0, CHUNK)])
        return carry

    lax.fori_loop(0, N_CHUNKS, chunk, 0)


@jax.jit
def kernel(rnatok, table):
    fused = _build_fused(table)
    fused2d = fused.reshape(VOCAB * MAXLEN, D_MODEL)
    tok_flat = rnatok.reshape(TOKENS)
    out = _sc_gather(tok_flat, fused2d)
    return out.reshape(B, MAXLEN, D_MODEL)


# speedup vs baseline: 2.3304x; 2.3304x over previous
import functools

import jax
import jax.numpy as jnp
import numpy as np
from jax import lax
from jax.experimental import pallas as pl
from jax.experimental.pallas import tpu as pltpu
from jax.experimental.pallas import tpu_sc as plsc

B = 64
MAXLEN = 2048
D_MODEL = 768
VOCAB = 5

NC = 2
NS = 16
LANES = 16
NW = NC * NS

TOKENS = B * MAXLEN
T_PER_W = TOKENS // NW
CHUNK = 128
N_CHUNKS = T_PER_W // CHUNK


def _pe_scaled() -> np.ndarray:
    pos = np.arange(MAXLEN)[:, np.newaxis]
    i = np.arange(D_MODEL)[np.newaxis, :]
    angle_rates = 1.0 / np.power(10000, 2 * (i // 2) / np.float32(D_MODEL))
    angle_rads = pos * angle_rates
    angle_rads[:, 0::2] = np.sin(angle_rads[:, 0::2])
    angle_rads[:, 1::2] = np.cos(angle_rads[:, 1::2])
    return (angle_rads / np.sqrt(np.float32(D_MODEL))).astype(np.float32)


_PE_SCALED = _pe_scaled()


def _fuse_body(table_ref, pe_ref, out_ref):
    out_ref[...] = table_ref[...][:, None, :] + pe_ref[...][None, :, :]


_L_BLK = 256


def _build_fused(table):
    pe = jnp.asarray(_PE_SCALED)
    return pl.pallas_call(
        _fuse_body,
        grid=(MAXLEN // _L_BLK,),
        in_specs=[
            pl.BlockSpec((VOCAB, D_MODEL), lambda i: (0, 0)),
            pl.BlockSpec((_L_BLK, D_MODEL), lambda i: (i, 0)),
        ],
        out_specs=pl.BlockSpec((VOCAB, _L_BLK, D_MODEL), lambda i: (0, i, 0)),
        out_shape=jax.ShapeDtypeStruct((VOCAB, MAXLEN, D_MODEL), jnp.float32),
    )(table, pe)


@functools.partial(
    pl.kernel,
    out_type=jax.ShapeDtypeStruct((TOKENS, D_MODEL), jnp.float32),
    mesh=plsc.VectorSubcoreMesh(core_axis_name="c", subcore_axis_name="s"),
    scratch_types=[
        pltpu.VMEM((CHUNK,), jnp.int32),
        pltpu.VMEM((CHUNK, D_MODEL), jnp.float32),
        pltpu.SemaphoreType.DMA,
    ],
)
def _sc_gather(tok_hbm, fused_hbm, out_hbm, idx_v, rows_v, sem):
    wid = lax.axis_index("s") * NC + lax.axis_index("c")
    base = wid * T_PER_W

    def chunk(i, carry):
        t0 = base + i * CHUNK
        pltpu.sync_copy(tok_hbm.at[pl.ds(t0, CHUNK)], idx_v)
        for r in range(CHUNK // LANES):
            sl = pl.ds(r * LANES, LANES)
            tok16 = idx_v[sl]
            t16 = (t0 + r * LANES) + lax.iota(jnp.int32, LANES)
            idx_v[sl] = (tok16 << 11) | (t16 & (MAXLEN - 1))
        pltpu.async_copy(fused_hbm.at[idx_v], rows_v, sem).wait()
        pltpu.sync_copy(rows_v, out_hbm.at[pl.ds(t0, CHUNK)])
        return carry

    lax.fori_loop(0, N_CHUNKS, chunk, 0)


@jax.jit
def kernel(rnatok, table):
    fused = _build_fused(table)
    fused2d = fused.reshape(VOCAB * MAXLEN, D_MODEL)
    tok_flat = rnatok.reshape(TOKENS)
    out = _sc_gather(tok_flat, fused2d)
    return out.reshape(B, MAXLEN, D_MODEL)


# speedup vs baseline: 2.4965x; 1.0713x over previous
import functools

import jax
import jax.numpy as jnp
import numpy as np
from jax import lax
from jax.experimental import pallas as pl
from jax.experimental.pallas import tpu as pltpu
from jax.experimental.pallas import tpu_sc as plsc

B = 64
MAXLEN = 2048
D_MODEL = 768
VOCAB = 5

NC = 2
NS = 16
LANES = 16
NW = NC * NS

TOKENS = B * MAXLEN
T_PER_W = TOKENS // NW
CHUNK = 64
N_CHUNKS = T_PER_W // CHUNK


def _pe_scaled() -> np.ndarray:
    pos = np.arange(MAXLEN)[:, np.newaxis]
    i = np.arange(D_MODEL)[np.newaxis, :]
    angle_rates = 1.0 / np.power(10000, 2 * (i // 2) / np.float32(D_MODEL))
    angle_rads = pos * angle_rates
    angle_rads[:, 0::2] = np.sin(angle_rads[:, 0::2])
    angle_rads[:, 1::2] = np.cos(angle_rads[:, 1::2])
    return (angle_rads / np.sqrt(np.float32(D_MODEL))).astype(np.float32)


_PE_SCALED = _pe_scaled()


def _fuse_body(table_ref, pe_ref, out_ref):
    out_ref[...] = table_ref[...][:, None, :] + pe_ref[...][None, :, :]


_L_BLK = 256


def _build_fused(table):
    pe = jnp.asarray(_PE_SCALED)
    return pl.pallas_call(
        _fuse_body,
        grid=(MAXLEN // _L_BLK,),
        in_specs=[
            pl.BlockSpec((VOCAB, D_MODEL), lambda i: (0, 0)),
            pl.BlockSpec((_L_BLK, D_MODEL), lambda i: (i, 0)),
        ],
        out_specs=pl.BlockSpec((VOCAB, _L_BLK, D_MODEL), lambda i: (0, i, 0)),
        out_shape=jax.ShapeDtypeStruct((VOCAB, MAXLEN, D_MODEL), jnp.float32),
    )(table, pe)


@functools.partial(
    pl.kernel,
    out_type=jax.ShapeDtypeStruct((TOKENS, D_MODEL), jnp.float32),
    mesh=plsc.VectorSubcoreMesh(core_axis_name="c", subcore_axis_name="s"),
    scratch_types=[
        pltpu.VMEM((CHUNK,), jnp.int32),
        pltpu.VMEM((CHUNK,), jnp.int32),
        pltpu.VMEM((CHUNK, D_MODEL), jnp.float32),
        pltpu.VMEM((CHUNK, D_MODEL), jnp.float32),
        pltpu.SemaphoreType.DMA,
        pltpu.SemaphoreType.DMA,
        pltpu.SemaphoreType.DMA,
        pltpu.SemaphoreType.DMA,
    ],
)
def _sc_gather(tok_hbm, fused_hbm, out_hbm,
               idx0, idx1, rows0, rows1, gsem0, gsem1, osem0, osem1):
    wid = lax.axis_index("s") * NC + lax.axis_index("c")
    base = wid * T_PER_W

    def fire_gather(c, idx_v, rows_v, sem):
        t0 = base + c * CHUNK
        pltpu.sync_copy(tok_hbm.at[pl.ds(t0, CHUNK)], idx_v)
        for r in range(CHUNK // LANES):
            sl = pl.ds(r * LANES, LANES)
            tok16 = idx_v[sl]
            t16 = (t0 + r * LANES) + lax.iota(jnp.int32, LANES)
            idx_v[sl] = (tok16 << 11) | (t16 & (MAXLEN - 1))
        pltpu.async_copy(fused_hbm.at[idx_v], rows_v, sem)

    def wait_gather(idx_v, rows_v, sem):
        pltpu.make_async_copy(fused_hbm.at[idx_v], rows_v, sem).wait()

    def fire_out(c, rows_v, sem):
        pltpu.async_copy(rows_v, out_hbm.at[pl.ds(base + c * CHUNK, CHUNK)], sem)

    def wait_out(c, rows_v, sem):
        pltpu.make_async_copy(
            rows_v, out_hbm.at[pl.ds(base + c * CHUNK, CHUNK)], sem).wait()

    fire_gather(0, idx0, rows0, gsem0)

    def group(g, carry):
        c0 = 2 * g
        c1 = c0 + 1
        wait_gather(idx0, rows0, gsem0)

        @pl.when(g > 0)
        def _():
            wait_out(c1 - 2, rows1, osem1)

        fire_gather(c1, idx1, rows1, gsem1)
        fire_out(c0, rows0, osem0)
        wait_gather(idx1, rows1, gsem1)
        wait_out(c0, rows0, osem0)

        @pl.when(c0 + 2 < N_CHUNKS)
        def _():
            fire_gather(c0 + 2, idx0, rows0, gsem0)

        fire_out(c1, rows1, osem1)
        return carry

    lax.fori_loop(0, N_CHUNKS // 2, group, 0)
    wait_out(N_CHUNKS - 1, rows1, osem1)


@jax.jit
def kernel(rnatok, table):
    fused = _build_fused(table)
    fused2d = fused.reshape(VOCAB * MAXLEN, D_MODEL)
    tok_flat = rnatok.reshape(TOKENS)
    out = _sc_gather(tok_flat, fused2d)
    return out.reshape(B, MAXLEN, D_MODEL)


# speedup vs baseline: 2.5295x; 1.0132x over previous
import functools

import jax
import jax.numpy as jnp
import numpy as np
from jax import lax
from jax.experimental import pallas as pl
from jax.experimental.pallas import tpu as pltpu
from jax.experimental.pallas import tpu_sc as plsc

B = 64
MAXLEN = 2048
D_MODEL = 768
VOCAB = 5

NC = 2
NS = 16
LANES = 16
NW = NC * NS

TOKENS = B * MAXLEN
T_PER_W = TOKENS // NW
CHUNK = 64
N_CHUNKS = T_PER_W // CHUNK


def _pe_scaled() -> np.ndarray:
    pos = np.arange(MAXLEN)[:, np.newaxis]
    i = np.arange(D_MODEL)[np.newaxis, :]
    angle_rates = 1.0 / np.power(10000, 2 * (i // 2) / np.float32(D_MODEL))
    angle_rads = pos * angle_rates
    angle_rads[:, 0::2] = np.sin(angle_rads[:, 0::2])
    angle_rads[:, 1::2] = np.cos(angle_rads[:, 1::2])
    return (angle_rads / np.sqrt(np.float32(D_MODEL))).astype(np.float32)


_PE_SCALED = _pe_scaled()


def _fuse_body(table_ref, pe_ref, out_ref):
    out_ref[...] = table_ref[...][:, None, :] + pe_ref[...][None, :, :]


_L_BLK = 256


def _build_fused(table):
    pe = jnp.asarray(_PE_SCALED)
    return pl.pallas_call(
        _fuse_body,
        grid=(MAXLEN // _L_BLK,),
        in_specs=[
            pl.BlockSpec((VOCAB, D_MODEL), lambda i: (0, 0)),
            pl.BlockSpec((_L_BLK, D_MODEL), lambda i: (i, 0)),
        ],
        out_specs=pl.BlockSpec((VOCAB, _L_BLK, D_MODEL), lambda i: (0, i, 0)),
        out_shape=jax.ShapeDtypeStruct((VOCAB, MAXLEN, D_MODEL), jnp.float32),
    )(table, pe)


@functools.partial(
    pl.kernel,
    out_type=jax.ShapeDtypeStruct((TOKENS, D_MODEL), jnp.float32),
    mesh=plsc.VectorSubcoreMesh(core_axis_name="c", subcore_axis_name="s"),
    scratch_types=[
        pltpu.VMEM((T_PER_W,), jnp.int32),
        pltpu.VMEM((CHUNK, D_MODEL), jnp.float32),
        pltpu.VMEM((CHUNK, D_MODEL), jnp.float32),
        pltpu.SemaphoreType.DMA,
        pltpu.SemaphoreType.DMA,
        pltpu.SemaphoreType.DMA,
        pltpu.SemaphoreType.DMA,
    ],
)
def _sc_gather(tok_hbm, fused_hbm, out_hbm,
               idx_all, rows0, rows1, gsem0, gsem1, osem0, osem1):
    wid = lax.axis_index("s") * NC + lax.axis_index("c")
    base = wid * T_PER_W

    pltpu.sync_copy(tok_hbm.at[pl.ds(base, T_PER_W)], idx_all)

    def to_idx(r, carry):
        sl = pl.ds(r * LANES, LANES)
        tok16 = idx_all[sl]
        t16 = (base + r * LANES) + lax.iota(jnp.int32, LANES)
        idx_all[sl] = (tok16 << 11) | (t16 & (MAXLEN - 1))
        return carry

    lax.fori_loop(0, T_PER_W // LANES, to_idx, 0)

    def fire_gather(c, rows_v, sem):
        pltpu.async_copy(
            fused_hbm.at[idx_all.at[pl.ds(c * CHUNK, CHUNK)]], rows_v, sem)

    def wait_gather(c, rows_v, sem):
        pltpu.make_async_copy(
            fused_hbm.at[idx_all.at[pl.ds(c * CHUNK, CHUNK)]], rows_v, sem).wait()

    def fire_out(c, rows_v, sem):
        pltpu.async_copy(rows_v, out_hbm.at[pl.ds(base + c * CHUNK, CHUNK)], sem)

    def wait_out(c, rows_v, sem):
        pltpu.make_async_copy(
            rows_v, out_hbm.at[pl.ds(base + c * CHUNK, CHUNK)], sem).wait()

    fire_gather(0, rows0, gsem0)

    def group(g, carry):
        c0 = 2 * g
        c1 = c0 + 1
        wait_gather(c0, rows0, gsem0)

        @pl.when(g > 0)
        def _():
            wait_out(c1 - 2, rows1, osem1)

        fire_gather(c1, rows1, gsem1)
        fire_out(c0, rows0, osem0)
        wait_gather(c1, rows1, gsem1)
        wait_out(c0, rows0, osem0)

        @pl.when(c0 + 2 < N_CHUNKS)
        def _():
            fire_gather(c0 + 2, rows0, gsem0)

        fire_out(c1, rows1, osem1)
        return carry

    lax.fori_loop(0, N_CHUNKS // 2, group, 0)
    wait_out(N_CHUNKS - 1, rows1, osem1)


@jax.jit
def kernel(rnatok, table):
    fused = _build_fused(table)
    fused2d = fused.reshape(VOCAB * MAXLEN, D_MODEL)
    tok_flat = rnatok.reshape(TOKENS)
    out = _sc_gather(tok_flat, fused2d)
    return out.reshape(B, MAXLEN, D_MODEL)
